# baseline (speedup 1.0000x reference)
import functools

import jax
import jax.numpy as jnp
from jax import lax
from jax.experimental import pallas as pl
from jax.experimental.pallas import tpu as pltpu
from jax.experimental.pallas import tpu_sc as plsc

BATCH = 4096
SEQ_LEN = 200
EMB = 64
TOTAL = BATCH * SEQ_LEN

NC = 2
NS = 16
NW = NC * NS
ROWS_PER_W = TOTAL // NW
CHUNK = 128
NCHUNK = ROWS_PER_W // CHUNK


def _make_embed():
    mesh = plsc.VectorSubcoreMesh(core_axis_name="c", subcore_axis_name="s")

    @functools.partial(
        pl.kernel,
        mesh=mesh,
        out_type=jax.ShapeDtypeStruct((TOTAL, EMB), jnp.float32),
        scratch_types=[
            pltpu.VMEM((NCHUNK, CHUNK), jnp.int32),
            pltpu.VMEM((CHUNK, EMB), jnp.float32),
            pltpu.SemaphoreType.DMA,
        ],
        compiler_params=pltpu.CompilerParams(use_tc_tiling_on_sc=False),
    )
    def embed(table_hbm, idx_hbm, out_hbm, idx_v, rows_v, sem):
        wid = lax.axis_index("s") * NC + lax.axis_index("c")
        base = wid * ROWS_PER_W
        pltpu.sync_copy(idx_hbm.at[wid], idx_v)

        def chunk_body(j, carry):
            pltpu.async_copy(table_hbm.at[idx_v.at[j]], rows_v, sem).wait()
            pltpu.sync_copy(rows_v, out_hbm.at[pl.ds(base + j * CHUNK, CHUNK)])
            return carry

        lax.fori_loop(0, NCHUNK, chunk_body, 0)

    return embed


_embed = _make_embed()


def kernel(inputs, embedding_table):
    idx = inputs.reshape(NW, NCHUNK, CHUNK).astype(jnp.int32)
    flat = _embed(embedding_table, idx)
    return flat.reshape(BATCH, SEQ_LEN, EMB)


# speedup vs baseline: 1.1144x; 1.1144x over previous
import functools

import jax
import jax.numpy as jnp
from jax import lax
from jax.experimental import pallas as pl
from jax.experimental.pallas import tpu as pltpu
from jax.experimental.pallas import tpu_sc as plsc

BATCH = 4096
SEQ_LEN = 200
EMB = 64
TOTAL = BATCH * SEQ_LEN

NC = 2
NS = 16
NW = NC * NS
ROWS_PER_W = TOTAL // NW
LANE = 128
NROW = ROWS_PER_W // LANE
K = 4
NCH = NROW // K
NBUF = 2


def _make_embed():
    mesh = plsc.VectorSubcoreMesh(core_axis_name="c", subcore_axis_name="s")

    @functools.partial(
        pl.kernel,
        mesh=mesh,
        out_type=jax.ShapeDtypeStruct((TOTAL // LANE, LANE, EMB), jnp.float32),
        scratch_types=[
            pltpu.VMEM((NROW, LANE), jnp.int32),
            pltpu.VMEM((NBUF, K, LANE, EMB), jnp.float32),
            pltpu.SemaphoreType.DMA((NBUF,)),
            pltpu.SemaphoreType.DMA((NBUF,)),
        ],
        compiler_params=pltpu.CompilerParams(use_tc_tiling_on_sc=False),
    )
    def embed(table_hbm, idx_hbm, out_hbm, idx_v, bufs, gsem, wsem):
        wid = lax.axis_index("s") * NC + lax.axis_index("c")
        base = wid * NROW
        pltpu.sync_copy(idx_hbm.at[wid], idx_v)

        def _gather_part(c, b, kk):
            return pltpu.make_async_copy(
                table_hbm.at[idx_v.at[c * K + kk]], bufs.at[b, kk], gsem.at[b]
            )

        class _Gather:
            def __init__(self, c, b):
                self.c, self.b = c, b

            def start(self):
                for kk in range(K):
                    _gather_part(self.c, self.b, kk).start()

            def wait(self):
                for kk in range(K):
                    _gather_part(self.c, self.b, kk).wait()

        def gather(c, b):
            return _Gather(c, b)

        def write(c, b):
            return pltpu.make_async_copy(
                bufs.at[b], out_hbm.at[pl.ds(base + c * K, K)], wsem.at[b]
            )

        for b in range(NBUF):
            gather(b, b).start()

        def round_body(r, carry):
            for b in range(NBUF):
                c = r * NBUF + b
                gather(c, b).wait()
                write(c, b).start()
                write(c, b).wait()

                @pl.when(c + NBUF < NCH)
                def _():
                    gather(c + NBUF, b).start()

            return carry

        lax.fori_loop(0, NCH // NBUF, round_body, 0)

    return embed


_embed = _make_embed()


def kernel(inputs, embedding_table):
    idx = inputs.reshape(NW, NROW, LANE).astype(jnp.int32)
    out = _embed(embedding_table, idx)
    return out.reshape(BATCH, SEQ_LEN, EMB)


# speedup vs baseline: 1.1479x; 1.0301x over previous
import functools

import jax
import jax.numpy as jnp
from jax import lax
from jax.experimental import pallas as pl
from jax.experimental.pallas import tpu as pltpu
from jax.experimental.pallas import tpu_sc as plsc

BATCH = 4096
SEQ_LEN = 200
EMB = 64
TOTAL = BATCH * SEQ_LEN

NC = 2
NS = 16
NW = NC * NS
ROWS_PER_W = TOTAL // NW
CHUNK = 128
NCH = ROWS_PER_W // CHUNK
NBUF = 4
OUT_W = TOTAL * EMB // 128
OW_PER_CH = CHUNK * EMB // 128


def _make_embed():
    mesh = plsc.VectorSubcoreMesh(core_axis_name="c", subcore_axis_name="s")

    @functools.partial(
        pl.kernel,
        mesh=mesh,
        out_type=jax.ShapeDtypeStruct((TOTAL, EMB), jnp.float32),
        scratch_types=[
            pltpu.VMEM((ROWS_PER_W,), jnp.int32),
            pltpu.VMEM((NBUF, CHUNK, EMB), jnp.float32),
            pltpu.SemaphoreType.DMA((NBUF,)),
            pltpu.SemaphoreType.DMA((NBUF,)),
        ],
        compiler_params=pltpu.CompilerParams(use_tc_tiling_on_sc=False),
    )
    def embed(table_hbm, idx_hbm, out_hbm, idx_v, bufs, gsem, wsem):
        wid = lax.axis_index("s") * NC + lax.axis_index("c")
        pltpu.sync_copy(idx_hbm.at[pl.ds(wid * ROWS_PER_W, ROWS_PER_W)], idx_v)
        obase = wid * ROWS_PER_W

        def gather(c, b):
            return pltpu.make_async_copy(
                table_hbm.at[idx_v.at[pl.ds(c * CHUNK, CHUNK)]],
                bufs.at[b],
                gsem.at[b],
            )

        def write(c, b):
            return pltpu.make_async_copy(
                bufs.at[b],
                out_hbm.at[pl.ds(obase + c * CHUNK, CHUNK)],
                wsem.at[b],
            )

        for b in range(NBUF):
            gather(b, b).start()

        def round_body(r, carry):
            for b in range(NBUF):
                c = r * NBUF + b
                gather(c, b).wait()
                write(c, b).start()
                write(c, b).wait()

                @pl.when(c + NBUF < NCH)
                def _():
                    gather(c + NBUF, b).start()

            return carry

        lax.fori_loop(0, NCH // NBUF, round_body, 0)

    return embed


_embed = _make_embed()


def kernel(inputs, embedding_table):
    idx = jnp.transpose(inputs).reshape(TOTAL).astype(jnp.int32)
    outw = _embed(embedding_table, idx)
    out_sm = outw.reshape(SEQ_LEN, BATCH, EMB)
    return jnp.transpose(out_sm, (1, 0, 2))
